# Initial kernel scaffold; baseline (speedup 1.0000x reference)
import jax
import jax.numpy as jnp
from jax.experimental import pallas as pl


def kernel(inputs, targets, curvatures):
    raise NotImplementedError("write your pallas kernel here")


# speedup vs baseline: 37.0630x; 37.0630x over previous
import jax
import jax.numpy as jnp
from jax import lax
from jax.experimental import pallas as pl
from jax.experimental.pallas import tpu as pltpu
from jax.experimental.pallas import tpu_sc as plsc

N = 16 * 512 * 512
K = int(0.4 * N)
NBITS = 12
NB = 1 << NBITS
NC, NS, L = 2, 16, 16
NW = NC * NS
PER_W = N // NW
CH = 8192
NCHUNK = PER_W // CH
SL = NB // NS

_MESH = plsc.VectorSubcoreMesh(core_axis_name="c", subcore_axis_name="s")


def _hist_body(c_hbm, x_hbm, t_hbm, hist_out,
               cbuf, xbuf, tbuf, h0, h1, h2, shared, tmp, acc):
    cc = lax.axis_index("c")
    ss = lax.axis_index("s")
    w = cc * NS + ss
    base = w * PER_W
    zero16 = jnp.zeros((L,), jnp.float32)
    ones16 = jnp.ones((L,), jnp.float32)

    def zero_hists(i, _):
        h0[pl.ds(i * L, L)] = zero16
        h1[pl.ds(i * L, L)] = zero16
        h2[pl.ds(i * L, L)] = zero16
        return 0
    lax.fori_loop(0, NB // L, zero_hists, 0)

    def chunk(g, _):
        off = base + g * CH
        pltpu.sync_copy(c_hbm.at[pl.ds(off, CH)], cbuf)
        pltpu.sync_copy(x_hbm.at[pl.ds(off, CH)], xbuf)
        pltpu.sync_copy(t_hbm.at[pl.ds(off, CH)], tbuf)

        def step(i, _):
            cv = cbuf[pl.ds(i * L, L)]
            xv = xbuf[pl.ds(i * L, L)]
            tv = tbuf[pl.ds(i * L, L)]
            ci = lax.bitcast_convert_type(cv, jnp.int32)
            sgn = lax.shift_right_arithmetic(ci, 31)
            key = lax.bitwise_xor(
                ci, lax.bitwise_or(sgn, jnp.int32(-(2 ** 31))))
            b = lax.shift_right_logical(key, 32 - NBITS)
            sig = 1.0 / (1.0 + jnp.exp(-xv))
            plsc.addupdate_scatter(h0, [b], ones16)
            plsc.addupdate_scatter(h1, [b], sig * tv)
            plsc.addupdate_scatter(h2, [b], sig + tv)
            return 0
        lax.fori_loop(0, CH // L, step, 0)
        return 0
    lax.fori_loop(0, NCHUNK, chunk, 0)

    pltpu.sync_copy(h0, shared.at[pl.ds((ss * 3 + 0) * NB, NB)])
    pltpu.sync_copy(h1, shared.at[pl.ds((ss * 3 + 1) * NB, NB)])
    pltpu.sync_copy(h2, shared.at[pl.ds((ss * 3 + 2) * NB, NB)])
    plsc.subcore_barrier()

    def zero_acc(i, _):
        acc[pl.ds(i * L, L)] = zero16
        return 0
    lax.fori_loop(0, 3 * SL // L, zero_acc, 0)

    def reduce_tile(j, _):
        for h in range(3):
            pltpu.sync_copy(shared.at[pl.ds((j * 3 + h) * NB + ss * SL, SL)],
                            tmp.at[pl.ds(h * SL, SL)])

        def addv(i, _):
            acc[pl.ds(i * L, L)] = acc[pl.ds(i * L, L)] + tmp[pl.ds(i * L, L)]
            return 0
        lax.fori_loop(0, 3 * SL // L, addv, 0)
        return 0
    lax.fori_loop(0, NS, reduce_tile, 0)

    for h in range(3):
        pltpu.sync_copy(acc.at[pl.ds(h * SL, SL)],
                        hist_out.at[pl.ds((cc * 3 + h) * NB + ss * SL, SL)])


def _final_body(hist_hbm, out_hbm, loc, comb, outv):
    cc = lax.axis_index("c")
    ss = lax.axis_index("s")

    @pl.when(jnp.logical_and(cc == 0, ss == 0))
    def _():
        pltpu.sync_copy(hist_hbm, loc)

        def combine(i, _):
            comb[pl.ds(i * L, L)] = (loc[pl.ds(i * L, L)]
                                     + loc[pl.ds(3 * NB + i * L, L)])
            return 0
        lax.fori_loop(0, 3 * NB // L, combine, 0)

        kf = jnp.float32(K)
        iota16 = lax.iota(jnp.int32, 16)

        def find(it, carry):
            run, bstar = carry
            i = NB // L - 1 - it
            v = comb[pl.ds(i * L, L)]
            sfx = lax.rev(plsc.cumsum(lax.rev(v, (0,))), (0,))
            gs = sfx + run
            mask = jnp.logical_and(gs >= kf, run < kf)
            cand = jnp.where(mask, iota16 + i * L, jnp.int32(-1))
            bstar = jnp.maximum(bstar, jnp.max(cand))
            run = run + jnp.sum(v)
            return run, bstar
        _, bstar = lax.fori_loop(0, NB // L, find,
                                 (jnp.float32(0.0), jnp.int32(-1)))

        zero16 = jnp.zeros((L,), jnp.float32)

        def accum(i, carry):
            ca, cb, ia, ib, sa, sb = carry
            lane = iota16 + i * L
            v0 = comb[pl.ds(i * L, L)]
            v1 = comb[pl.ds(NB + i * L, L)]
            v2 = comb[pl.ds(2 * NB + i * L, L)]
            m_above = lane > bstar
            m_bin = lane == bstar
            ca = ca + jnp.where(m_above, v0, 0.0)
            cb = cb + jnp.where(m_bin, v0, 0.0)
            ia = ia + jnp.where(m_above, v1, 0.0)
            ib = ib + jnp.where(m_bin, v1, 0.0)
            sa = sa + jnp.where(m_above, v2, 0.0)
            sb = sb + jnp.where(m_bin, v2, 0.0)
            return ca, cb, ia, ib, sa, sb
        ca, cb, ia, ib, sa, sb = lax.fori_loop(
            0, NB // L, accum, (zero16,) * 6)

        count_above_v = zero16 + jnp.sum(ca)
        t_bin_v = zero16 + jnp.sum(cb)
        frac_v = (kf - count_above_v) / t_bin_v
        inter_v = (zero16 + jnp.sum(ia)) + frac_v * (zero16 + jnp.sum(ib))
        sums_v = (zero16 + jnp.sum(sa)) + frac_v * (zero16 + jnp.sum(sb))
        loss_v = 1.0 - (2.0 * inter_v + 1.0) / (sums_v + 1.0)
        outv[...] = loss_v
        pltpu.sync_copy(outv, out_hbm)


_hist_call = pl.kernel(
    _hist_body,
    out_type=jax.ShapeDtypeStruct((NC * 3 * NB,), jnp.float32),
    mesh=_MESH,
    compiler_params=pltpu.CompilerParams(needs_layout_passes=False),
    scratch_types=[
        pltpu.VMEM((CH,), jnp.float32),
        pltpu.VMEM((CH,), jnp.float32),
        pltpu.VMEM((CH,), jnp.float32),
        pltpu.VMEM((NB,), jnp.float32),
        pltpu.VMEM((NB,), jnp.float32),
        pltpu.VMEM((NB,), jnp.float32),
        pltpu.VMEM_SHARED((NS * 3 * NB,), jnp.float32),
        pltpu.VMEM((3 * SL,), jnp.float32),
        pltpu.VMEM((3 * SL,), jnp.float32),
    ],
)

_final_call = pl.kernel(
    _final_body,
    out_type=jax.ShapeDtypeStruct((L,), jnp.float32),
    mesh=_MESH,
    compiler_params=pltpu.CompilerParams(needs_layout_passes=False),
    scratch_types=[
        pltpu.VMEM((NC * 3 * NB,), jnp.float32),
        pltpu.VMEM((3 * NB,), jnp.float32),
        pltpu.VMEM((L,), jnp.float32),
    ],
)


@jax.jit
def kernel(inputs, targets, curvatures):
    x = inputs.reshape(-1)
    t = targets.reshape(-1)
    c = curvatures.reshape(-1)
    hist = _hist_call(c, x, t)
    out = _final_call(hist)
    return out[0]


# speedup vs baseline: 42.6958x; 1.1520x over previous
import jax
import jax.numpy as jnp
from jax import lax
from jax.experimental import pallas as pl
from jax.experimental.pallas import tpu as pltpu
from jax.experimental.pallas import tpu_sc as plsc

N = 16 * 512 * 512
K = int(0.4 * N)
NBITS = 12
NB = 1 << NBITS
NC, NS, L = 2, 16, 16
NW = NC * NS
PER_W = N // NW
CH = 8192
NCHUNK = PER_W // CH
SL = NB // NS

_MESH = plsc.VectorSubcoreMesh(core_axis_name="c", subcore_axis_name="s")


UF = 4


def _hist_body(c_hbm, x_hbm, t_hbm, hist_out,
               cbuf_a, xbuf_a, tbuf_a, cbuf_b, xbuf_b, tbuf_b,
               h0, h1, h2, shared, tmp, acc, sem_a, sem_b):
    cc = lax.axis_index("c")
    ss = lax.axis_index("s")
    w = cc * NS + ss
    base = w * PER_W
    zero16 = jnp.zeros((L,), jnp.float32)
    ones16 = jnp.ones((L,), jnp.float32)

    def zero_hists(i, _):
        h0[pl.ds(i * L, L)] = zero16
        h1[pl.ds(i * L, L)] = zero16
        h2[pl.ds(i * L, L)] = zero16
        return 0
    lax.fori_loop(0, NB // L, zero_hists, 0)

    def start(g, bufs, sem):
        off = base + g * CH
        pltpu.async_copy(c_hbm.at[pl.ds(off, CH)], bufs[0], sem)
        pltpu.async_copy(x_hbm.at[pl.ds(off, CH)], bufs[1], sem)
        pltpu.async_copy(t_hbm.at[pl.ds(off, CH)], bufs[2], sem)

    def wait(bufs, sem):
        for b in bufs:
            pltpu.make_async_copy(c_hbm.at[pl.ds(0, CH)], b, sem).wait()

    def process(bufs):
        cb, xb, tb = bufs

        def step(i, _):
            for j in range(UF):
                o = (i * UF + j) * L
                cv = cb[pl.ds(o, L)]
                xv = xb[pl.ds(o, L)]
                tv = tb[pl.ds(o, L)]
                ci = lax.bitcast_convert_type(cv, jnp.int32)
                sgn = lax.shift_right_arithmetic(ci, 31)
                key = lax.bitwise_xor(
                    ci, lax.bitwise_or(sgn, jnp.int32(-(2 ** 31))))
                b = lax.shift_right_logical(key, 32 - NBITS)
                sig = 1.0 / (1.0 + jnp.exp(-xv))
                plsc.addupdate_scatter(h0, [b], ones16)
                plsc.addupdate_scatter(h1, [b], sig * tv)
                plsc.addupdate_scatter(h2, [b], sig + tv)
            return 0
        lax.fori_loop(0, CH // L // UF, step, 0)

    bufs_a = (cbuf_a, xbuf_a, tbuf_a)
    bufs_b = (cbuf_b, xbuf_b, tbuf_b)

    start(0, bufs_a, sem_a)

    def chunk_pair(g, _):
        start(2 * g + 1, bufs_b, sem_b)
        wait(bufs_a, sem_a)
        process(bufs_a)

        @pl.when(2 * g + 2 < NCHUNK)
        def _():
            start(2 * g + 2, bufs_a, sem_a)
        wait(bufs_b, sem_b)
        process(bufs_b)
        return 0
    lax.fori_loop(0, NCHUNK // 2, chunk_pair, 0)

    pltpu.sync_copy(h0, shared.at[pl.ds((ss * 3 + 0) * NB, NB)])
    pltpu.sync_copy(h1, shared.at[pl.ds((ss * 3 + 1) * NB, NB)])
    pltpu.sync_copy(h2, shared.at[pl.ds((ss * 3 + 2) * NB, NB)])
    plsc.subcore_barrier()

    def zero_acc(i, _):
        acc[pl.ds(i * L, L)] = zero16
        return 0
    lax.fori_loop(0, 3 * SL // L, zero_acc, 0)

    def reduce_tile(j, _):
        for h in range(3):
            pltpu.sync_copy(shared.at[pl.ds((j * 3 + h) * NB + ss * SL, SL)],
                            tmp.at[pl.ds(h * SL, SL)])

        def addv(i, _):
            acc[pl.ds(i * L, L)] = acc[pl.ds(i * L, L)] + tmp[pl.ds(i * L, L)]
            return 0
        lax.fori_loop(0, 3 * SL // L, addv, 0)
        return 0
    lax.fori_loop(0, NS, reduce_tile, 0)

    for h in range(3):
        pltpu.sync_copy(acc.at[pl.ds(h * SL, SL)],
                        hist_out.at[pl.ds((cc * 3 + h) * NB + ss * SL, SL)])


def _final_body(hist_hbm, out_hbm, loc, comb, outv):
    cc = lax.axis_index("c")
    ss = lax.axis_index("s")

    @pl.when(jnp.logical_and(cc == 0, ss == 0))
    def _():
        pltpu.sync_copy(hist_hbm, loc)

        def combine(i, _):
            comb[pl.ds(i * L, L)] = (loc[pl.ds(i * L, L)]
                                     + loc[pl.ds(3 * NB + i * L, L)])
            return 0
        lax.fori_loop(0, 3 * NB // L, combine, 0)

        kf = jnp.float32(K)
        iota16 = lax.iota(jnp.int32, 16)

        def find(it, carry):
            run, bstar = carry
            i = NB // L - 1 - it
            v = comb[pl.ds(i * L, L)]
            sfx = lax.rev(plsc.cumsum(lax.rev(v, (0,))), (0,))
            gs = sfx + run
            mask = jnp.logical_and(gs >= kf, run < kf)
            cand = jnp.where(mask, iota16 + i * L, jnp.int32(-1))
            bstar = jnp.maximum(bstar, jnp.max(cand))
            run = run + jnp.sum(v)
            return run, bstar
        _, bstar = lax.fori_loop(0, NB // L, find,
                                 (jnp.float32(0.0), jnp.int32(-1)))

        zero16 = jnp.zeros((L,), jnp.float32)

        def accum(i, carry):
            ca, cb, ia, ib, sa, sb = carry
            lane = iota16 + i * L
            v0 = comb[pl.ds(i * L, L)]
            v1 = comb[pl.ds(NB + i * L, L)]
            v2 = comb[pl.ds(2 * NB + i * L, L)]
            m_above = lane > bstar
            m_bin = lane == bstar
            ca = ca + jnp.where(m_above, v0, 0.0)
            cb = cb + jnp.where(m_bin, v0, 0.0)
            ia = ia + jnp.where(m_above, v1, 0.0)
            ib = ib + jnp.where(m_bin, v1, 0.0)
            sa = sa + jnp.where(m_above, v2, 0.0)
            sb = sb + jnp.where(m_bin, v2, 0.0)
            return ca, cb, ia, ib, sa, sb
        ca, cb, ia, ib, sa, sb = lax.fori_loop(
            0, NB // L, accum, (zero16,) * 6)

        count_above_v = zero16 + jnp.sum(ca)
        t_bin_v = zero16 + jnp.sum(cb)
        frac_v = (kf - count_above_v) / t_bin_v
        inter_v = (zero16 + jnp.sum(ia)) + frac_v * (zero16 + jnp.sum(ib))
        sums_v = (zero16 + jnp.sum(sa)) + frac_v * (zero16 + jnp.sum(sb))
        loss_v = 1.0 - (2.0 * inter_v + 1.0) / (sums_v + 1.0)
        outv[...] = loss_v
        pltpu.sync_copy(outv, out_hbm)


_hist_call = pl.kernel(
    _hist_body,
    out_type=jax.ShapeDtypeStruct((NC * 3 * NB,), jnp.float32),
    mesh=_MESH,
    compiler_params=pltpu.CompilerParams(needs_layout_passes=False),
    scratch_types=[
        pltpu.VMEM((CH,), jnp.float32),
        pltpu.VMEM((CH,), jnp.float32),
        pltpu.VMEM((CH,), jnp.float32),
        pltpu.VMEM((CH,), jnp.float32),
        pltpu.VMEM((CH,), jnp.float32),
        pltpu.VMEM((CH,), jnp.float32),
        pltpu.VMEM((NB,), jnp.float32),
        pltpu.VMEM((NB,), jnp.float32),
        pltpu.VMEM((NB,), jnp.float32),
        pltpu.VMEM_SHARED((NS * 3 * NB,), jnp.float32),
        pltpu.VMEM((3 * SL,), jnp.float32),
        pltpu.VMEM((3 * SL,), jnp.float32),
        pltpu.SemaphoreType.DMA,
        pltpu.SemaphoreType.DMA,
    ],
)

_final_call = pl.kernel(
    _final_body,
    out_type=jax.ShapeDtypeStruct((L,), jnp.float32),
    mesh=_MESH,
    compiler_params=pltpu.CompilerParams(needs_layout_passes=False),
    scratch_types=[
        pltpu.VMEM((NC * 3 * NB,), jnp.float32),
        pltpu.VMEM((3 * NB,), jnp.float32),
        pltpu.VMEM((L,), jnp.float32),
    ],
)


@jax.jit
def kernel(inputs, targets, curvatures):
    x = inputs.reshape(-1)
    t = targets.reshape(-1)
    c = curvatures.reshape(-1)
    hist = _hist_call(c, x, t)
    out = _final_call(hist)
    return out[0]


# speedup vs baseline: 73.1632x; 1.7136x over previous
import jax
import jax.numpy as jnp
from jax import lax
from jax.experimental import pallas as pl
from jax.experimental.pallas import tpu as pltpu
from jax.experimental.pallas import tpu_sc as plsc

N = 16 * 512 * 512
K = int(0.4 * N)
NBITS = 12
NB = 1 << NBITS
NC, NS, L = 2, 16, 16
NW = NC * NS
PER_W = N // NW
CH = 8192
NCHUNK = PER_W // CH
SL = NB // NS

_MESH = plsc.VectorSubcoreMesh(core_axis_name="c", subcore_axis_name="s")


UF = 4


def _hist_body(c_hbm, x_hbm, t_hbm, hist_out,
               cbuf_a, xbuf_a, tbuf_a, cbuf_b, xbuf_b, tbuf_b,
               h0, h1, h2, shared, tmp, acc, sem_a, sem_b):
    cc = lax.axis_index("c")
    ss = lax.axis_index("s")
    w = cc * NS + ss
    base = w * PER_W
    zero16 = jnp.zeros((L,), jnp.float32)
    ones16 = jnp.ones((L,), jnp.float32)

    def zero_hists(i, _):
        h0[pl.ds(i * L, L)] = zero16
        h1[pl.ds(i * L, L)] = zero16
        h2[pl.ds(i * L, L)] = zero16
        return 0
    lax.fori_loop(0, NB // L, zero_hists, 0)

    def start(g, bufs, sem):
        off = base + g * CH
        pltpu.async_copy(c_hbm.at[pl.ds(off, CH)], bufs[0], sem)
        pltpu.async_copy(x_hbm.at[pl.ds(off, CH)], bufs[1], sem)
        pltpu.async_copy(t_hbm.at[pl.ds(off, CH)], bufs[2], sem)

    def wait(bufs, sem):
        for b in bufs:
            pltpu.make_async_copy(c_hbm.at[pl.ds(0, CH)], b, sem).wait()

    def process(bufs):
        cb, xb, tb = bufs

        def step(i, _):
            offs = [(i * UF + j) * L for j in range(UF)]
            cvs = [cb[pl.ds(o, L)] for o in offs]
            xvs = [xb[pl.ds(o, L)] for o in offs]
            tvs = [tb[pl.ds(o, L)] for o in offs]
            bins = []
            for j in range(UF):
                ci = lax.bitcast_convert_type(cvs[j], jnp.int32)
                sgn = lax.shift_right_arithmetic(ci, 31)
                key = lax.bitwise_xor(
                    ci, lax.bitwise_or(sgn, jnp.int32(-(2 ** 31))))
                bins.append(lax.shift_right_logical(key, 32 - NBITS))
            sigs = [1.0 / (1.0 + jnp.exp(-xvs[j])) for j in range(UF)]
            for j in range(UF):
                plsc.addupdate_scatter(h0, [bins[j]], ones16)
                plsc.addupdate_scatter(h1, [bins[j]], sigs[j] * tvs[j])
                plsc.addupdate_scatter(h2, [bins[j]], sigs[j] + tvs[j])
            return 0
        lax.fori_loop(0, CH // L // UF, step, 0)

    bufs_a = (cbuf_a, xbuf_a, tbuf_a)
    bufs_b = (cbuf_b, xbuf_b, tbuf_b)

    start(0, bufs_a, sem_a)

    def chunk_pair(g, _):
        start(2 * g + 1, bufs_b, sem_b)
        wait(bufs_a, sem_a)
        process(bufs_a)

        @pl.when(2 * g + 2 < NCHUNK)
        def _():
            start(2 * g + 2, bufs_a, sem_a)
        wait(bufs_b, sem_b)
        process(bufs_b)
        return 0
    lax.fori_loop(0, NCHUNK // 2, chunk_pair, 0)

    pltpu.sync_copy(h0, shared.at[pl.ds((ss * 3 + 0) * NB, NB)])
    pltpu.sync_copy(h1, shared.at[pl.ds((ss * 3 + 1) * NB, NB)])
    pltpu.sync_copy(h2, shared.at[pl.ds((ss * 3 + 2) * NB, NB)])
    plsc.subcore_barrier()

    def zero_acc(i, _):
        acc[pl.ds(i * L, L)] = zero16
        return 0
    lax.fori_loop(0, 3 * SL // L, zero_acc, 0)

    def reduce_tile(j, _):
        for h in range(3):
            pltpu.sync_copy(shared.at[pl.ds((j * 3 + h) * NB + ss * SL, SL)],
                            tmp.at[pl.ds(h * SL, SL)])

        def addv(i, _):
            acc[pl.ds(i * L, L)] = acc[pl.ds(i * L, L)] + tmp[pl.ds(i * L, L)]
            return 0
        lax.fori_loop(0, 3 * SL // L, addv, 0)
        return 0
    lax.fori_loop(0, NS, reduce_tile, 0)

    for h in range(3):
        pltpu.sync_copy(acc.at[pl.ds(h * SL, SL)],
                        hist_out.at[pl.ds((cc * 3 + h) * NB + ss * SL, SL)])


def _final_body(hist_hbm, out_hbm, loc, comb, outv):
    cc = lax.axis_index("c")
    ss = lax.axis_index("s")

    @pl.when(jnp.logical_and(cc == 0, ss == 0))
    def _():
        pltpu.sync_copy(hist_hbm, loc)

        def combine(i, _):
            comb[pl.ds(i * L, L)] = (loc[pl.ds(i * L, L)]
                                     + loc[pl.ds(3 * NB + i * L, L)])
            return 0
        lax.fori_loop(0, 3 * NB // L, combine, 0)

        kf = jnp.float32(K)
        iota16 = lax.iota(jnp.int32, 16)

        def find(it, carry):
            run, bstar = carry
            i = NB // L - 1 - it
            v = comb[pl.ds(i * L, L)]
            sfx = lax.rev(plsc.cumsum(lax.rev(v, (0,))), (0,))
            gs = sfx + run
            mask = jnp.logical_and(gs >= kf, run < kf)
            cand = jnp.where(mask, iota16 + i * L, jnp.int32(-1))
            bstar = jnp.maximum(bstar, jnp.max(cand))
            run = run + jnp.sum(v)
            return run, bstar
        _, bstar = lax.fori_loop(0, NB // L, find,
                                 (jnp.float32(0.0), jnp.int32(-1)))

        zero16 = jnp.zeros((L,), jnp.float32)

        def accum(i, carry):
            ca, cb, ia, ib, sa, sb = carry
            lane = iota16 + i * L
            v0 = comb[pl.ds(i * L, L)]
            v1 = comb[pl.ds(NB + i * L, L)]
            v2 = comb[pl.ds(2 * NB + i * L, L)]
            m_above = lane > bstar
            m_bin = lane == bstar
            ca = ca + jnp.where(m_above, v0, 0.0)
            cb = cb + jnp.where(m_bin, v0, 0.0)
            ia = ia + jnp.where(m_above, v1, 0.0)
            ib = ib + jnp.where(m_bin, v1, 0.0)
            sa = sa + jnp.where(m_above, v2, 0.0)
            sb = sb + jnp.where(m_bin, v2, 0.0)
            return ca, cb, ia, ib, sa, sb
        ca, cb, ia, ib, sa, sb = lax.fori_loop(
            0, NB // L, accum, (zero16,) * 6)

        count_above_v = zero16 + jnp.sum(ca)
        t_bin_v = zero16 + jnp.sum(cb)
        frac_v = (kf - count_above_v) / t_bin_v
        inter_v = (zero16 + jnp.sum(ia)) + frac_v * (zero16 + jnp.sum(ib))
        sums_v = (zero16 + jnp.sum(sa)) + frac_v * (zero16 + jnp.sum(sb))
        loss_v = 1.0 - (2.0 * inter_v + 1.0) / (sums_v + 1.0)
        outv[...] = loss_v
        pltpu.sync_copy(outv, out_hbm)


_hist_call = pl.kernel(
    _hist_body,
    out_type=jax.ShapeDtypeStruct((NC * 3 * NB,), jnp.float32),
    mesh=_MESH,
    compiler_params=pltpu.CompilerParams(needs_layout_passes=False),
    scratch_types=[
        pltpu.VMEM((CH,), jnp.float32),
        pltpu.VMEM((CH,), jnp.float32),
        pltpu.VMEM((CH,), jnp.float32),
        pltpu.VMEM((CH,), jnp.float32),
        pltpu.VMEM((CH,), jnp.float32),
        pltpu.VMEM((CH,), jnp.float32),
        pltpu.VMEM((NB,), jnp.float32),
        pltpu.VMEM((NB,), jnp.float32),
        pltpu.VMEM((NB,), jnp.float32),
        pltpu.VMEM_SHARED((NS * 3 * NB,), jnp.float32),
        pltpu.VMEM((3 * SL,), jnp.float32),
        pltpu.VMEM((3 * SL,), jnp.float32),
        pltpu.SemaphoreType.DMA,
        pltpu.SemaphoreType.DMA,
    ],
)

_final_call = pl.kernel(
    _final_body,
    out_type=jax.ShapeDtypeStruct((L,), jnp.float32),
    mesh=_MESH,
    compiler_params=pltpu.CompilerParams(needs_layout_passes=False),
    scratch_types=[
        pltpu.VMEM((NC * 3 * NB,), jnp.float32),
        pltpu.VMEM((3 * NB,), jnp.float32),
        pltpu.VMEM((L,), jnp.float32),
    ],
)


@jax.jit
def kernel(inputs, targets, curvatures):
    x = inputs.reshape(-1)
    t = targets.reshape(-1)
    c = curvatures.reshape(-1)
    hist = _hist_call(c, x, t)
    out = _final_call(hist)
    return out[0]


# speedup vs baseline: 82.1137x; 1.1223x over previous
import jax
import jax.numpy as jnp
from jax import lax
from jax.experimental import pallas as pl
from jax.experimental.pallas import tpu as pltpu
from jax.experimental.pallas import tpu_sc as plsc

N = 16 * 512 * 512
K = int(0.4 * N)
NBITS = 12
NB = 1 << NBITS
NC, NS, L = 2, 16, 16
NW = NC * NS
PER_W = N // NW
CH = 8192
NCHUNK = PER_W // CH
SL = NB // NS

_MESH = plsc.VectorSubcoreMesh(core_axis_name="c", subcore_axis_name="s")


UF = 8


def _hist_body(c_hbm, x_hbm, t_hbm, hist_out,
               cbuf_a, xbuf_a, tbuf_a, cbuf_b, xbuf_b, tbuf_b,
               h0, h1, h2, shared, tmp, acc, sem_a, sem_b):
    cc = lax.axis_index("c")
    ss = lax.axis_index("s")
    w = cc * NS + ss
    base = w * PER_W
    zero16 = jnp.zeros((L,), jnp.float32)
    ones16 = jnp.ones((L,), jnp.float32)

    def zero_hists(i, _):
        h0[pl.ds(i * L, L)] = zero16
        h1[pl.ds(i * L, L)] = zero16
        h2[pl.ds(i * L, L)] = zero16
        return 0
    lax.fori_loop(0, NB // L, zero_hists, 0)

    def start(g, bufs, sem):
        off = base + g * CH
        pltpu.async_copy(c_hbm.at[pl.ds(off, CH)], bufs[0], sem)
        pltpu.async_copy(x_hbm.at[pl.ds(off, CH)], bufs[1], sem)
        pltpu.async_copy(t_hbm.at[pl.ds(off, CH)], bufs[2], sem)

    def wait(bufs, sem):
        for b in bufs:
            pltpu.make_async_copy(c_hbm.at[pl.ds(0, CH)], b, sem).wait()

    def process(bufs):
        cb, xb, tb = bufs

        def step(i, _):
            offs = [(i * UF + j) * L for j in range(UF)]
            cvs = [cb[pl.ds(o, L)] for o in offs]
            xvs = [xb[pl.ds(o, L)] for o in offs]
            tvs = [tb[pl.ds(o, L)] for o in offs]
            bins = []
            for j in range(UF):
                ci = lax.bitcast_convert_type(cvs[j], jnp.int32)
                sgn = lax.shift_right_arithmetic(ci, 31)
                key = lax.bitwise_xor(
                    ci, lax.bitwise_or(sgn, jnp.int32(-(2 ** 31))))
                bins.append(lax.shift_right_logical(key, 32 - NBITS))
            sigs = [1.0 / (1.0 + jnp.exp(-xvs[j])) for j in range(UF)]
            for j in range(UF):
                plsc.addupdate_scatter(h0, [bins[j]], ones16)
                plsc.addupdate_scatter(h1, [bins[j]], sigs[j] * tvs[j])
                plsc.addupdate_scatter(h2, [bins[j]], sigs[j] + tvs[j])
            return 0
        lax.fori_loop(0, CH // L // UF, step, 0)

    bufs_a = (cbuf_a, xbuf_a, tbuf_a)
    bufs_b = (cbuf_b, xbuf_b, tbuf_b)

    start(0, bufs_a, sem_a)

    def chunk_pair(g, _):
        start(2 * g + 1, bufs_b, sem_b)
        wait(bufs_a, sem_a)
        process(bufs_a)

        @pl.when(2 * g + 2 < NCHUNK)
        def _():
            start(2 * g + 2, bufs_a, sem_a)
        wait(bufs_b, sem_b)
        process(bufs_b)
        return 0
    lax.fori_loop(0, NCHUNK // 2, chunk_pair, 0)

    pltpu.sync_copy(h0, shared.at[pl.ds((ss * 3 + 0) * NB, NB)])
    pltpu.sync_copy(h1, shared.at[pl.ds((ss * 3 + 1) * NB, NB)])
    pltpu.sync_copy(h2, shared.at[pl.ds((ss * 3 + 2) * NB, NB)])
    plsc.subcore_barrier()

    def zero_acc(i, _):
        acc[pl.ds(i * L, L)] = zero16
        return 0
    lax.fori_loop(0, 3 * SL // L, zero_acc, 0)

    def reduce_tile(j, _):
        for h in range(3):
            pltpu.sync_copy(shared.at[pl.ds((j * 3 + h) * NB + ss * SL, SL)],
                            tmp.at[pl.ds(h * SL, SL)])

        def addv(i, _):
            acc[pl.ds(i * L, L)] = acc[pl.ds(i * L, L)] + tmp[pl.ds(i * L, L)]
            return 0
        lax.fori_loop(0, 3 * SL // L, addv, 0)
        return 0
    lax.fori_loop(0, NS, reduce_tile, 0)

    for h in range(3):
        pltpu.sync_copy(acc.at[pl.ds(h * SL, SL)],
                        hist_out.at[pl.ds((cc * 3 + h) * NB + ss * SL, SL)])


def _final_body(hist_hbm, out_hbm, loc, comb, outv):
    cc = lax.axis_index("c")
    ss = lax.axis_index("s")

    @pl.when(jnp.logical_and(cc == 0, ss == 0))
    def _():
        pltpu.sync_copy(hist_hbm, loc)

        def combine(i, _):
            comb[pl.ds(i * L, L)] = (loc[pl.ds(i * L, L)]
                                     + loc[pl.ds(3 * NB + i * L, L)])
            return 0
        lax.fori_loop(0, 3 * NB // L, combine, 0)

        kf = jnp.float32(K)
        iota16 = lax.iota(jnp.int32, 16)

        def find(it, carry):
            run, bstar = carry
            i = NB // L - 1 - it
            v = comb[pl.ds(i * L, L)]
            sfx = lax.rev(plsc.cumsum(lax.rev(v, (0,))), (0,))
            gs = sfx + run
            mask = jnp.logical_and(gs >= kf, run < kf)
            cand = jnp.where(mask, iota16 + i * L, jnp.int32(-1))
            bstar = jnp.maximum(bstar, jnp.max(cand))
            run = run + jnp.sum(v)
            return run, bstar
        _, bstar = lax.fori_loop(0, NB // L, find,
                                 (jnp.float32(0.0), jnp.int32(-1)))

        zero16 = jnp.zeros((L,), jnp.float32)

        def accum(i, carry):
            ca, cb, ia, ib, sa, sb = carry
            lane = iota16 + i * L
            v0 = comb[pl.ds(i * L, L)]
            v1 = comb[pl.ds(NB + i * L, L)]
            v2 = comb[pl.ds(2 * NB + i * L, L)]
            m_above = lane > bstar
            m_bin = lane == bstar
            ca = ca + jnp.where(m_above, v0, 0.0)
            cb = cb + jnp.where(m_bin, v0, 0.0)
            ia = ia + jnp.where(m_above, v1, 0.0)
            ib = ib + jnp.where(m_bin, v1, 0.0)
            sa = sa + jnp.where(m_above, v2, 0.0)
            sb = sb + jnp.where(m_bin, v2, 0.0)
            return ca, cb, ia, ib, sa, sb
        ca, cb, ia, ib, sa, sb = lax.fori_loop(
            0, NB // L, accum, (zero16,) * 6)

        count_above_v = zero16 + jnp.sum(ca)
        t_bin_v = zero16 + jnp.sum(cb)
        frac_v = (kf - count_above_v) / t_bin_v
        inter_v = (zero16 + jnp.sum(ia)) + frac_v * (zero16 + jnp.sum(ib))
        sums_v = (zero16 + jnp.sum(sa)) + frac_v * (zero16 + jnp.sum(sb))
        loss_v = 1.0 - (2.0 * inter_v + 1.0) / (sums_v + 1.0)
        outv[...] = loss_v
        pltpu.sync_copy(outv, out_hbm)


_hist_call = pl.kernel(
    _hist_body,
    out_type=jax.ShapeDtypeStruct((NC * 3 * NB,), jnp.float32),
    mesh=_MESH,
    compiler_params=pltpu.CompilerParams(needs_layout_passes=False),
    scratch_types=[
        pltpu.VMEM((CH,), jnp.float32),
        pltpu.VMEM((CH,), jnp.float32),
        pltpu.VMEM((CH,), jnp.float32),
        pltpu.VMEM((CH,), jnp.float32),
        pltpu.VMEM((CH,), jnp.float32),
        pltpu.VMEM((CH,), jnp.float32),
        pltpu.VMEM((NB,), jnp.float32),
        pltpu.VMEM((NB,), jnp.float32),
        pltpu.VMEM((NB,), jnp.float32),
        pltpu.VMEM_SHARED((NS * 3 * NB,), jnp.float32),
        pltpu.VMEM((3 * SL,), jnp.float32),
        pltpu.VMEM((3 * SL,), jnp.float32),
        pltpu.SemaphoreType.DMA,
        pltpu.SemaphoreType.DMA,
    ],
)

_final_call = pl.kernel(
    _final_body,
    out_type=jax.ShapeDtypeStruct((L,), jnp.float32),
    mesh=_MESH,
    compiler_params=pltpu.CompilerParams(needs_layout_passes=False),
    scratch_types=[
        pltpu.VMEM((NC * 3 * NB,), jnp.float32),
        pltpu.VMEM((3 * NB,), jnp.float32),
        pltpu.VMEM((L,), jnp.float32),
    ],
)


@jax.jit
def kernel(inputs, targets, curvatures):
    x = inputs.reshape(-1)
    t = targets.reshape(-1)
    c = curvatures.reshape(-1)
    hist = _hist_call(c, x, t)
    out = _final_call(hist)
    return out[0]


# speedup vs baseline: 123.2496x; 1.5010x over previous
import jax
import jax.numpy as jnp
from jax import lax
from jax.experimental import pallas as pl
from jax.experimental.pallas import tpu as pltpu
from jax.experimental.pallas import tpu_sc as plsc

N = 16 * 512 * 512
K = int(0.4 * N)
NBITS = 12
NB = 1 << NBITS
NC, NS, L = 2, 16, 16
NW = NC * NS
PER_W = N // NW
CH = 8192
NCHUNK = PER_W // CH
SL = NB // NS

_MESH = plsc.VectorSubcoreMesh(core_axis_name="c", subcore_axis_name="s")


UF = 8
CROWS = 16


def _hist_body(c_hbm, x_hbm, t_hbm, hist_out,
               cbuf_a, xbuf_a, tbuf_a, cbuf_b, xbuf_b, tbuf_b,
               h0, h1, h2, shared, tmp, acc, sem_a, sem_b):
    cc = lax.axis_index("c")
    ss = lax.axis_index("s")
    w = cc * NS + ss
    batch = w // 2
    row0 = (w % 2) * 256
    zero16 = jnp.zeros((L,), jnp.float32)
    ones16 = jnp.ones((L,), jnp.float32)

    def zero_hists(i, _):
        h0[pl.ds(i * L, L)] = zero16
        h1[pl.ds(i * L, L)] = zero16
        h2[pl.ds(i * L, L)] = zero16
        return 0
    lax.fori_loop(0, NB // L, zero_hists, 0)

    def start(g, bufs, sem):
        r = row0 + g * CROWS
        pltpu.async_copy(c_hbm.at[batch, 0, pl.ds(r, CROWS), :], bufs[0], sem)
        pltpu.async_copy(x_hbm.at[batch, 0, pl.ds(r, CROWS), :], bufs[1], sem)
        pltpu.async_copy(t_hbm.at[batch, 0, pl.ds(r, CROWS), :], bufs[2], sem)

    def wait(bufs, sem):
        for b in bufs:
            pltpu.make_async_copy(
                c_hbm.at[0, 0, pl.ds(0, CROWS), :], b, sem).wait()

    def process(bufs):
        cb, xb, tb = bufs

        def step(i, _):
            v0 = i * UF
            offs = [((v0 + j) // 32, ((v0 + j) % 32) * L) for j in range(UF)]
            cvs = [cb[r, pl.ds(o, L)] for (r, o) in offs]
            xvs = [xb[r, pl.ds(o, L)] for (r, o) in offs]
            tvs = [tb[r, pl.ds(o, L)] for (r, o) in offs]
            bins = []
            for j in range(UF):
                ci = lax.bitcast_convert_type(cvs[j], jnp.int32)
                sgn = lax.shift_right_arithmetic(ci, 31)
                key = lax.bitwise_xor(
                    ci, lax.bitwise_or(sgn, jnp.int32(-(2 ** 31))))
                bins.append(lax.shift_right_logical(key, 32 - NBITS))
            sigs = [1.0 / (1.0 + jnp.exp(-xvs[j])) for j in range(UF)]
            for j in range(UF):
                plsc.addupdate_scatter(h0, [bins[j]], ones16)
                plsc.addupdate_scatter(h1, [bins[j]], sigs[j] * tvs[j])
                plsc.addupdate_scatter(h2, [bins[j]], sigs[j] + tvs[j])
            return 0
        lax.fori_loop(0, CH // L // UF, step, 0)

    bufs_a = (cbuf_a, xbuf_a, tbuf_a)
    bufs_b = (cbuf_b, xbuf_b, tbuf_b)

    start(0, bufs_a, sem_a)

    def chunk_pair(g, _):
        start(2 * g + 1, bufs_b, sem_b)
        wait(bufs_a, sem_a)
        process(bufs_a)

        @pl.when(2 * g + 2 < NCHUNK)
        def _():
            start(2 * g + 2, bufs_a, sem_a)
        wait(bufs_b, sem_b)
        process(bufs_b)
        return 0
    lax.fori_loop(0, NCHUNK // 2, chunk_pair, 0)

    pltpu.sync_copy(h0, shared.at[pl.ds((ss * 3 + 0) * NB, NB)])
    pltpu.sync_copy(h1, shared.at[pl.ds((ss * 3 + 1) * NB, NB)])
    pltpu.sync_copy(h2, shared.at[pl.ds((ss * 3 + 2) * NB, NB)])
    plsc.subcore_barrier()

    def zero_acc(i, _):
        acc[pl.ds(i * L, L)] = zero16
        return 0
    lax.fori_loop(0, 3 * SL // L, zero_acc, 0)

    def reduce_tile(j, _):
        for h in range(3):
            pltpu.sync_copy(shared.at[pl.ds((j * 3 + h) * NB + ss * SL, SL)],
                            tmp.at[pl.ds(h * SL, SL)])

        def addv(i, _):
            acc[pl.ds(i * L, L)] = acc[pl.ds(i * L, L)] + tmp[pl.ds(i * L, L)]
            return 0
        lax.fori_loop(0, 3 * SL // L, addv, 0)
        return 0
    lax.fori_loop(0, NS, reduce_tile, 0)

    for h in range(3):
        pltpu.sync_copy(acc.at[pl.ds(h * SL, SL)],
                        hist_out.at[pl.ds((cc * 3 + h) * NB + ss * SL, SL)])


def _final_body(hist_hbm, out_hbm, loc, comb, outv):
    cc = lax.axis_index("c")
    ss = lax.axis_index("s")

    @pl.when(jnp.logical_and(cc == 0, ss == 0))
    def _():
        pltpu.sync_copy(hist_hbm, loc)

        def combine(i, _):
            comb[pl.ds(i * L, L)] = (loc[pl.ds(i * L, L)]
                                     + loc[pl.ds(3 * NB + i * L, L)])
            return 0
        lax.fori_loop(0, 3 * NB // L, combine, 0)

        kf = jnp.float32(K)
        iota16 = lax.iota(jnp.int32, 16)

        def find(it, carry):
            run, bstar = carry
            i = NB // L - 1 - it
            v = comb[pl.ds(i * L, L)]
            sfx = lax.rev(plsc.cumsum(lax.rev(v, (0,))), (0,))
            gs = sfx + run
            mask = jnp.logical_and(gs >= kf, run < kf)
            cand = jnp.where(mask, iota16 + i * L, jnp.int32(-1))
            bstar = jnp.maximum(bstar, jnp.max(cand))
            run = run + jnp.sum(v)
            return run, bstar
        _, bstar = lax.fori_loop(0, NB // L, find,
                                 (jnp.float32(0.0), jnp.int32(-1)))

        zero16 = jnp.zeros((L,), jnp.float32)

        def accum(i, carry):
            ca, cb, ia, ib, sa, sb = carry
            lane = iota16 + i * L
            v0 = comb[pl.ds(i * L, L)]
            v1 = comb[pl.ds(NB + i * L, L)]
            v2 = comb[pl.ds(2 * NB + i * L, L)]
            m_above = lane > bstar
            m_bin = lane == bstar
            ca = ca + jnp.where(m_above, v0, 0.0)
            cb = cb + jnp.where(m_bin, v0, 0.0)
            ia = ia + jnp.where(m_above, v1, 0.0)
            ib = ib + jnp.where(m_bin, v1, 0.0)
            sa = sa + jnp.where(m_above, v2, 0.0)
            sb = sb + jnp.where(m_bin, v2, 0.0)
            return ca, cb, ia, ib, sa, sb
        ca, cb, ia, ib, sa, sb = lax.fori_loop(
            0, NB // L, accum, (zero16,) * 6)

        count_above_v = zero16 + jnp.sum(ca)
        t_bin_v = zero16 + jnp.sum(cb)
        frac_v = (kf - count_above_v) / t_bin_v
        inter_v = (zero16 + jnp.sum(ia)) + frac_v * (zero16 + jnp.sum(ib))
        sums_v = (zero16 + jnp.sum(sa)) + frac_v * (zero16 + jnp.sum(sb))
        loss_v = 1.0 - (2.0 * inter_v + 1.0) / (sums_v + 1.0)
        outv[...] = loss_v
        pltpu.sync_copy(outv, out_hbm)


_hist_call = pl.kernel(
    _hist_body,
    out_type=jax.ShapeDtypeStruct((NC * 3 * NB,), jnp.float32),
    mesh=_MESH,
    compiler_params=pltpu.CompilerParams(needs_layout_passes=False,
                                         use_tc_tiling_on_sc=True),
    scratch_types=[
        pltpu.VMEM((CROWS, 512), jnp.float32),
        pltpu.VMEM((CROWS, 512), jnp.float32),
        pltpu.VMEM((CROWS, 512), jnp.float32),
        pltpu.VMEM((CROWS, 512), jnp.float32),
        pltpu.VMEM((CROWS, 512), jnp.float32),
        pltpu.VMEM((CROWS, 512), jnp.float32),
        pltpu.VMEM((NB,), jnp.float32),
        pltpu.VMEM((NB,), jnp.float32),
        pltpu.VMEM((NB,), jnp.float32),
        pltpu.VMEM_SHARED((NS * 3 * NB,), jnp.float32),
        pltpu.VMEM((3 * SL,), jnp.float32),
        pltpu.VMEM((3 * SL,), jnp.float32),
        pltpu.SemaphoreType.DMA,
        pltpu.SemaphoreType.DMA,
    ],
)

_final_call = pl.kernel(
    _final_body,
    out_type=jax.ShapeDtypeStruct((L,), jnp.float32),
    mesh=_MESH,
    compiler_params=pltpu.CompilerParams(needs_layout_passes=False),
    scratch_types=[
        pltpu.VMEM((NC * 3 * NB,), jnp.float32),
        pltpu.VMEM((3 * NB,), jnp.float32),
        pltpu.VMEM((L,), jnp.float32),
    ],
)


@jax.jit
def kernel(inputs, targets, curvatures):
    hist = _hist_call(curvatures, inputs, targets)
    out = _final_call(hist)
    return out[0]
